# baseline (speedup 1.0000x reference)
import functools

import jax
import jax.numpy as jnp
from jax.experimental import pallas as pl


def _router_tile(x_ref, w1_ref, b1_ref, tb_ref, w2_ref, b2_ref, out_ref):
    h = jnp.dot(x_ref[...], w1_ref[...], preferred_element_type=jnp.float32)
    h = h + b1_ref[...]
    h = 0.5 * h * (1.0 + jax.lax.erf(h * 0.7071067811865476)) + tb_ref[...]
    logits = jnp.dot(h, w2_ref[...], preferred_element_type=jnp.float32)
    logits = logits + b2_ref[...]

    c = logits.shape[-1]
    iota = jax.lax.broadcasted_iota(jnp.int32, logits.shape, 1)
    m1 = jnp.max(logits, axis=-1, keepdims=True)
    idx1 = jnp.min(jnp.where(logits == m1, iota, c), axis=-1, keepdims=True)
    hit1 = iota == idx1
    masked = jnp.where(hit1, -jnp.inf, logits)
    m2 = jnp.max(masked, axis=-1, keepdims=True)
    idx2 = jnp.min(jnp.where(masked == m2, iota, c), axis=-1, keepdims=True)
    hit2 = iota == idx2
    e2 = jnp.exp(m2 - m1)
    p1 = 1.0 / (1.0 + e2)
    p2 = 1.0 - p1
    out_ref[...] = jnp.where(hit1, p1, jnp.where(hit2, p2, 0.0))


@functools.partial(jax.jit, static_argnames=())
def kernel(x, W1, b1, W2, b2, task_table, task_id):
    original_shape = x.shape
    xf = x.reshape(-1, x.shape[-1])
    n, d = xf.shape
    e = W1.shape[1]
    c = W2.shape[1]
    tb = task_table[task_id].reshape(1, e)

    tm = 512
    grid = (n // tm,)
    probs = pl.pallas_call(
        _router_tile,
        grid=grid,
        in_specs=[
            pl.BlockSpec((tm, d), lambda i: (i, 0)),
            pl.BlockSpec((d, e), lambda i: (0, 0)),
            pl.BlockSpec((1, e), lambda i: (0, 0)),
            pl.BlockSpec((1, e), lambda i: (0, 0)),
            pl.BlockSpec((e, c), lambda i: (0, 0)),
            pl.BlockSpec((1, c), lambda i: (0, 0)),
        ],
        out_specs=pl.BlockSpec((tm, c), lambda i: (i, 0)),
        out_shape=jax.ShapeDtypeStruct((n, c), jnp.float32),
    )(xf, W1, b1.reshape(1, e), tb, W2, b2.reshape(1, c))
    return probs.reshape(*original_shape[:-1], c)


# speedup vs baseline: 1.1926x; 1.1926x over previous
import functools

import jax
import jax.numpy as jnp
from jax.experimental import pallas as pl


def _router_tile(x_ref, w1_ref, b1_ref, tb_ref, w2_ref, b2_ref, out_ref):
    h = jnp.dot(x_ref[...], w1_ref[...], preferred_element_type=jnp.float32)
    h = h + b1_ref[...]
    h = 0.5 * h * (1.0 + jax.lax.erf(h * 0.7071067811865476)) + tb_ref[...]
    logits = jnp.dot(h, w2_ref[...], preferred_element_type=jnp.float32)
    logits = logits + b2_ref[...]

    c = logits.shape[-1]
    iota = jax.lax.broadcasted_iota(jnp.int32, logits.shape, 1)
    m1 = jnp.max(logits, axis=-1, keepdims=True)
    idx1 = jnp.min(jnp.where(logits == m1, iota, c), axis=-1, keepdims=True)
    hit1 = iota == idx1
    masked = jnp.where(hit1, -jnp.inf, logits)
    m2 = jnp.max(masked, axis=-1, keepdims=True)
    idx2 = jnp.min(jnp.where(masked == m2, iota, c), axis=-1, keepdims=True)
    hit2 = iota == idx2
    e2 = jnp.exp(m2 - m1)
    p1 = 1.0 / (1.0 + e2)
    p2 = 1.0 - p1
    out_ref[...] = jnp.where(hit1, p1, jnp.where(hit2, p2, 0.0))


@functools.partial(jax.jit, static_argnames=())
def kernel(x, W1, b1, W2, b2, task_table, task_id):
    original_shape = x.shape
    xf = x.reshape(-1, x.shape[-1])
    n, d = xf.shape
    e = W1.shape[1]
    c = W2.shape[1]
    tb = task_table[task_id].reshape(1, e)

    tm = 2048
    grid = (n // tm,)
    probs = pl.pallas_call(
        _router_tile,
        grid=grid,
        in_specs=[
            pl.BlockSpec((tm, d), lambda i: (i, 0)),
            pl.BlockSpec((d, e), lambda i: (0, 0)),
            pl.BlockSpec((1, e), lambda i: (0, 0)),
            pl.BlockSpec((1, e), lambda i: (0, 0)),
            pl.BlockSpec((e, c), lambda i: (0, 0)),
            pl.BlockSpec((1, c), lambda i: (0, 0)),
        ],
        out_specs=pl.BlockSpec((tm, c), lambda i: (i, 0)),
        out_shape=jax.ShapeDtypeStruct((n, c), jnp.float32),
    )(xf, W1, b1.reshape(1, e), tb, W2, b2.reshape(1, c))
    return probs.reshape(*original_shape[:-1], c)
